# baseline (device time: 223740 ns/iter reference)
import jax
import jax.numpy as jnp
from jax import lax
from jax.experimental import pallas as pl
from jax.experimental.pallas import tpu as pltpu

NZ = 4
C = 8


def kernel(x):
    m, n = x.shape
    ch = m // C
    xb = x.astype(jnp.bfloat16)

    def body(x_ref, out_ref, pbuf, sbuf, psend, precv, ssend, srecv):
        my_x = lax.axis_index("x")
        my_y = lax.axis_index("y")
        my_z = lax.axis_index("z")
        not_first = my_z > 0
        not_last = my_z < NZ - 1

        pbuf[:, :] = jnp.zeros((m, n), jnp.bfloat16)
        sbuf[:, :] = jnp.zeros((m, n), jnp.bfloat16)

        bsem = pltpu.get_barrier_semaphore()

        @pl.when(not_first)
        def _():
            pl.semaphore_signal(
                bsem, inc=1,
                device_id=(my_x, my_y, my_z - 1),
                device_id_type=pl.DeviceIdType.MESH,
            )

        @pl.when(not_last)
        def _():
            pl.semaphore_signal(
                bsem, inc=1,
                device_id=(my_x, my_y, my_z + 1),
                device_id_type=pl.DeviceIdType.MESH,
            )

        n_nbrs = not_first.astype(jnp.int32) + not_last.astype(jnp.int32)
        pl.semaphore_wait(bsem, n_nbrs)

        def prefix_desc(c):
            return pltpu.make_async_remote_copy(
                src_ref=pbuf.at[pl.ds(c * ch, ch), :],
                dst_ref=pbuf.at[pl.ds(c * ch, ch), :],
                send_sem=psend.at[c],
                recv_sem=precv.at[c],
                device_id=(my_x, my_y, my_z + 1),
                device_id_type=pl.DeviceIdType.MESH,
            )

        def suffix_desc(c):
            return pltpu.make_async_remote_copy(
                src_ref=sbuf.at[pl.ds(c * ch, ch), :],
                dst_ref=sbuf.at[pl.ds(c * ch, ch), :],
                send_sem=ssend.at[c],
                recv_sem=srecv.at[c],
                device_id=(my_x, my_y, my_z - 1),
                device_id_type=pl.DeviceIdType.MESH,
            )

        for c in range(C):
            rows = pl.ds(c * ch, ch)

            @pl.when(not_first)
            def _(c=c):
                prefix_desc(c).wait_recv()

            pbuf[rows, :] = pbuf[rows, :] + x_ref[rows, :]

            @pl.when(not_last)
            def _(c=c):
                prefix_desc(c).start()

            @pl.when(not_last)
            def _(c=c):
                suffix_desc(c).wait_recv()

            out_ref[rows, :] = pbuf[rows, :] + sbuf[rows, :]

            sbuf[rows, :] = sbuf[rows, :] + x_ref[rows, :]

            @pl.when(not_first)
            def _(c=c):
                suffix_desc(c).start()

        for c in range(C):
            @pl.when(not_last)
            def _(c=c):
                prefix_desc(c).wait_send()

            @pl.when(not_first)
            def _(c=c):
                suffix_desc(c).wait_send()

    return pl.pallas_call(
        body,
        out_shape=jax.ShapeDtypeStruct((m, n), jnp.bfloat16),
        in_specs=[pl.BlockSpec(memory_space=pltpu.VMEM)],
        out_specs=pl.BlockSpec(memory_space=pltpu.VMEM),
        scratch_shapes=[
            pltpu.VMEM((m, n), jnp.bfloat16),
            pltpu.VMEM((m, n), jnp.bfloat16),
            pltpu.SemaphoreType.DMA((C,)),
            pltpu.SemaphoreType.DMA((C,)),
            pltpu.SemaphoreType.DMA((C,)),
            pltpu.SemaphoreType.DMA((C,)),
        ],
        compiler_params=pltpu.CompilerParams(collective_id=0),
    )(xb)


# device time: 37825 ns/iter; 5.9151x vs baseline; 5.9151x over previous
import jax
import jax.numpy as jnp
from jax import lax
from jax.experimental import pallas as pl
from jax.experimental.pallas import tpu as pltpu

NZ = 4


def kernel(x):
    m, n = x.shape
    m2 = m // 2
    ch = m2 // NZ
    xb = x.astype(jnp.bfloat16)

    def body(x_ref, out_ref, rs_buf, rs_send, rs_recv, ag_send, ag_recv,
             xg_send, xg_recv):
        my_x = lax.axis_index("x")
        my_y = lax.axis_index("y")
        my_z = lax.axis_index("z")
        base = my_x * m2
        own = base + my_z * ch

        bsem = pltpu.get_barrier_semaphore()
        for k in range(1, NZ):
            pl.semaphore_signal(
                bsem, inc=1,
                device_id=(my_x, my_y, (my_z + k) % NZ),
                device_id_type=pl.DeviceIdType.MESH,
            )
        pl.semaphore_signal(
            bsem, inc=1,
            device_id=(1 - my_x, my_y, my_z),
            device_id_type=pl.DeviceIdType.MESH,
        )
        pl.semaphore_wait(bsem, NZ)

        rs = []
        for k in range(1, NZ):
            dst = (my_z + k) % NZ
            rdma = pltpu.make_async_remote_copy(
                src_ref=x_ref.at[pl.ds(base + dst * ch, ch), :],
                dst_ref=rs_buf.at[k - 1],
                send_sem=rs_send.at[k - 1],
                recv_sem=rs_recv.at[k - 1],
                device_id=(my_x, my_y, dst),
                device_id_type=pl.DeviceIdType.MESH,
            )
            rdma.start()
            rs.append(rdma)
        for rdma in rs:
            rdma.wait()

        acc = x_ref[pl.ds(own, ch), :].astype(jnp.float32)
        for k in range(1, NZ):
            acc += rs_buf[k - 1, :, :].astype(jnp.float32)
        out_ref[pl.ds(own, ch), :] = acc.astype(jnp.bfloat16)

        xg = []
        g0 = pltpu.make_async_remote_copy(
            src_ref=out_ref.at[pl.ds(own, ch), :],
            dst_ref=out_ref.at[pl.ds(own, ch), :],
            send_sem=xg_send.at[0],
            recv_sem=xg_recv.at[0],
            device_id=(1 - my_x, my_y, my_z),
            device_id_type=pl.DeviceIdType.MESH,
        )
        g0.start()
        xg.append(g0)

        ag = []
        for k in range(1, NZ):
            dst = (my_z + k) % NZ
            rdma = pltpu.make_async_remote_copy(
                src_ref=out_ref.at[pl.ds(own, ch), :],
                dst_ref=out_ref.at[pl.ds(own, ch), :],
                send_sem=ag_send.at[k - 1],
                recv_sem=ag_recv.at[k - 1],
                device_id=(my_x, my_y, dst),
                device_id_type=pl.DeviceIdType.MESH,
            )
            rdma.start()
            ag.append(rdma)

        for k in range(1, NZ):
            ag[k - 1].wait_recv()
            src_z = (my_z - k) % NZ
            rows = pl.ds(base + src_z * ch, ch)
            g = pltpu.make_async_remote_copy(
                src_ref=out_ref.at[rows, :],
                dst_ref=out_ref.at[rows, :],
                send_sem=xg_send.at[k],
                recv_sem=xg_recv.at[k],
                device_id=(1 - my_x, my_y, my_z),
                device_id_type=pl.DeviceIdType.MESH,
            )
            g.start()
            xg.append(g)

        for rdma in ag:
            rdma.wait_send()
        for g in xg:
            g.wait()

    return pl.pallas_call(
        body,
        out_shape=jax.ShapeDtypeStruct((m, n), jnp.bfloat16),
        in_specs=[pl.BlockSpec(memory_space=pltpu.VMEM)],
        out_specs=pl.BlockSpec(memory_space=pltpu.VMEM),
        scratch_shapes=[
            pltpu.VMEM((NZ - 1, ch, n), jnp.bfloat16),
            pltpu.SemaphoreType.DMA((NZ - 1,)),
            pltpu.SemaphoreType.DMA((NZ - 1,)),
            pltpu.SemaphoreType.DMA((NZ - 1,)),
            pltpu.SemaphoreType.DMA((NZ - 1,)),
            pltpu.SemaphoreType.DMA((NZ,)),
            pltpu.SemaphoreType.DMA((NZ,)),
        ],
        compiler_params=pltpu.CompilerParams(collective_id=0),
    )(xb)
